# Initial kernel scaffold; baseline (speedup 1.0000x reference)
import jax
import jax.numpy as jnp
from jax.experimental import pallas as pl


def kernel(embedding, word_embedding0, lookup_ids, word_lookup_ids, random_ids):
    raise NotImplementedError("write your pallas kernel here")


# speedup vs baseline: 1.4905x; 1.4905x over previous
import functools

import jax
import jax.numpy as jnp
from jax import lax
from jax.experimental import pallas as pl
from jax.experimental.pallas import tpu as pltpu
from jax.experimental.pallas import tpu_sc as plsc

_B = 4096
_L = 200
_D = 32
_N = _B * _L
_NC = 2
_NS = 16
_NW = _NC * _NS
_NPW = _N // _NW
_CH = 1024
_NCH = _NPW // _CH
_BPW = _B // _NW
_LN = 16

_PAD_ID = 1
_EOS_ID = 0


def _masks_from_idx(idx_ref, real_ref, eos_ref):
    ones = jnp.full((_LN,), 1.0, jnp.float32)
    zeros = jnp.zeros((_LN,), jnp.float32)

    def body(i, carry):
        s = pl.ds(i * _LN, _LN)
        v = idx_ref[s]
        real_ref[s] = jnp.where(v != _PAD_ID, ones, zeros)
        eos_ref[s] = jnp.where(v == _EOS_ID, ones, zeros)
        return carry

    lax.fori_loop(0, _CH // _LN, body, 0)


def _sc_body(emb, wemb, lid, wid_ids, rid,
             mat_out, real_out, eos_out, vec_out, rnd_out,
             idx_a, idx_b, rows_a, rows_b, widx, wrows,
             mreal, meos, sem_a, sem_b):
    w = lax.axis_index("s") * _NC + lax.axis_index("c")
    base = w * _NPW
    idxb = (idx_a, idx_b)
    rowsb = (rows_a, rows_b)
    sems = (sem_a, sem_b)

    wb = w * _BPW
    pltpu.sync_copy(wid_ids.at[pl.ds(wb, _BPW)], widx)
    pltpu.make_async_copy(wemb.at[widx], wrows, sem_a).start()
    pltpu.make_async_copy(wemb.at[widx], wrows, sem_a).wait()
    pltpu.sync_copy(wrows, vec_out.at[pl.ds(wb, _BPW)])

    def run_table(idx_hbm, out_hbm, with_masks):
        def start_gather(slot, chunk):
            off = base + chunk * _CH
            pltpu.sync_copy(idx_hbm.at[pl.ds(off, _CH)], idxb[slot])
            pltpu.make_async_copy(emb.at[idxb[slot]], rowsb[slot], sems[slot]).start()

        def finish_chunk(slot, chunk):
            pltpu.make_async_copy(emb.at[idxb[slot]], rowsb[slot], sems[slot]).wait()
            off = base + chunk * _CH
            if with_masks:
                _masks_from_idx(idxb[slot], mreal, meos)
                pltpu.sync_copy(mreal, real_out.at[pl.ds(off, _CH)])
                pltpu.sync_copy(meos, eos_out.at[pl.ds(off, _CH)])
            pltpu.sync_copy(rowsb[slot], out_hbm.at[pl.ds(off, _CH)])

        start_gather(0, 0)
        def outer(t, carry):
            for slot in range(2):
                g = t * 2 + slot
                start_gather(1 - slot, g + 1)
                finish_chunk(slot, g)
            return carry

        lax.fori_loop(0, (_NCH - 1) // 2, outer, 0)
        finish_chunk((_NCH - 1) % 2, _NCH - 1)

    run_table(lid, mat_out, True)
    run_table(rid, rnd_out, False)


_mesh = plsc.VectorSubcoreMesh(core_axis_name="c", subcore_axis_name="s")

_sc_kernel = functools.partial(
    pl.kernel,
    mesh=_mesh,
    compiler_params=pltpu.CompilerParams(use_tc_tiling_on_sc=False),
    out_type=[
        jax.ShapeDtypeStruct((_N, _D), jnp.float32),
        jax.ShapeDtypeStruct((_N,), jnp.float32),
        jax.ShapeDtypeStruct((_N,), jnp.float32),
        jax.ShapeDtypeStruct((_B, _D), jnp.float32),
        jax.ShapeDtypeStruct((_N, _D), jnp.float32),
    ],
    scratch_types=[
        pltpu.VMEM((_CH,), jnp.int32),
        pltpu.VMEM((_CH,), jnp.int32),
        pltpu.VMEM((_CH, _D), jnp.float32),
        pltpu.VMEM((_CH, _D), jnp.float32),
        pltpu.VMEM((_BPW,), jnp.int32),
        pltpu.VMEM((_BPW, _D), jnp.float32),
        pltpu.VMEM((_CH,), jnp.float32),
        pltpu.VMEM((_CH,), jnp.float32),
        pltpu.SemaphoreType.DMA,
        pltpu.SemaphoreType.DMA,
    ],
)(_sc_body)


@jax.jit
def kernel(embedding, word_embedding0, lookup_ids, word_lookup_ids, random_ids):
    lid = lookup_ids.reshape(_N).astype(jnp.int32)
    rid = random_ids.reshape(_N).astype(jnp.int32)
    wid = word_lookup_ids.astype(jnp.int32)
    mat, real, eos, vec, rnd = _sc_kernel(embedding, word_embedding0, lid, wid, rid)
    return (
        mat.reshape(_B, _L, _D),
        real.reshape(_B, _L),
        eos.reshape(_B, _L),
        vec,
        rnd.reshape(_B, _L, _D),
    )


# speedup vs baseline: 1.8873x; 1.2662x over previous
import functools

import jax
import jax.numpy as jnp
from jax import lax
from jax.experimental import pallas as pl
from jax.experimental.pallas import tpu as pltpu
from jax.experimental.pallas import tpu_sc as plsc

_VOCAB = 1000000
_B = 4096
_L = 200
_D = 32
_N = _B * _L
_NC = 2
_NS = 16
_NW = _NC * _NS
_NPW = _N // _NW
_CH = 1024
_NCH = _NPW // _CH
_BPW = _B // _NW
_LN = 16

_PAD_ID = 1
_EOS_ID = 0


def _masks_from_idx(idx_ref, real_ref, eos_ref):
    ones = jnp.full((_LN,), 1.0, jnp.float32)
    zeros = jnp.zeros((_LN,), jnp.float32)

    def body(i, carry):
        s = pl.ds(i * _LN, _LN)
        v = idx_ref[s]
        real_ref[s] = jnp.where(v != _PAD_ID, ones, zeros)
        eos_ref[s] = jnp.where(v == _EOS_ID, ones, zeros)
        return carry

    lax.fori_loop(0, _CH // _LN, body, 0)


def _sc_body(emb, lid, rid,
             mat_out, real_out, eos_out, rnd_out,
             idx_a, idx_b, rows_a, rows_b,
             mreal, meos, sem_a, sem_b):
    w = lax.axis_index("s") * _NC + lax.axis_index("c")
    base = w * _NPW
    idxb = (idx_a, idx_b)
    rowsb = (rows_a, rows_b)
    sems = (sem_a, sem_b)

    def run_table(idx_hbm, out_hbm, with_masks):
        def start_gather(slot, chunk):
            off = base + chunk * _CH
            pltpu.sync_copy(idx_hbm.at[pl.ds(off, _CH)], idxb[slot])
            pltpu.make_async_copy(emb.at[idxb[slot]], rowsb[slot], sems[slot]).start()

        def finish_chunk(slot, chunk):
            pltpu.make_async_copy(emb.at[idxb[slot]], rowsb[slot], sems[slot]).wait()
            off = base + chunk * _CH
            if with_masks:
                _masks_from_idx(idxb[slot], mreal, meos)
                pltpu.sync_copy(mreal, real_out.at[pl.ds(off, _CH)])
                pltpu.sync_copy(meos, eos_out.at[pl.ds(off, _CH)])
            pltpu.sync_copy(rowsb[slot], out_hbm.at[pl.ds(off, _CH)])

        start_gather(0, 0)
        def outer(t, carry):
            for slot in range(2):
                g = t * 2 + slot
                start_gather(1 - slot, g + 1)
                finish_chunk(slot, g)
            return carry

        lax.fori_loop(0, (_NCH - 1) // 2, outer, 0)
        finish_chunk((_NCH - 1) % 2, _NCH - 1)

    run_table(lid, mat_out, True)
    run_table(rid, rnd_out, False)


_mesh = plsc.VectorSubcoreMesh(core_axis_name="c", subcore_axis_name="s")

_sc_kernel = functools.partial(
    pl.kernel,
    mesh=_mesh,
    compiler_params=pltpu.CompilerParams(use_tc_tiling_on_sc=False),
    out_type=[
        jax.ShapeDtypeStruct((_N, _D), jnp.float32),
        jax.ShapeDtypeStruct((_N,), jnp.float32),
        jax.ShapeDtypeStruct((_N,), jnp.float32),
        jax.ShapeDtypeStruct((_N, _D), jnp.float32),
    ],
    scratch_types=[
        pltpu.VMEM((_CH,), jnp.int32),
        pltpu.VMEM((_CH,), jnp.int32),
        pltpu.VMEM((_CH, _D), jnp.float32),
        pltpu.VMEM((_CH, _D), jnp.float32),
        pltpu.VMEM((_CH,), jnp.float32),
        pltpu.VMEM((_CH,), jnp.float32),
        pltpu.SemaphoreType.DMA,
        pltpu.SemaphoreType.DMA,
    ],
)(_sc_body)


def _vec_body(wembT, wid_ids, vecT_out, idx_full, tiles, colbuf, sem):
    w = lax.axis_index("s") * _NC + lax.axis_index("c")
    wb = pl.multiple_of(w * _BPW, 128)
    pltpu.sync_copy(wid_ids, idx_full)
    lanes = lax.iota(jnp.int32, 16)

    def group(j, carry):
        off = pl.multiple_of(wb + j * 16, 16)
        vec16 = idx_full[pl.ds(off, 16)]
        vids = [jnp.sum(jnp.where(lanes == k, vec16, 0)) for k in range(16)]
        bases = [pl.multiple_of(v - v % 128, 128) for v in vids]
        for k in range(16):
            pltpu.make_async_copy(
                wembT.at[:, pl.ds(bases[k], 128)], tiles.at[k], sem
            ).start()
        for k in range(16):
            pltpu.make_async_copy(
                wembT.at[:, pl.ds(bases[k], 128)], tiles.at[k], sem
            ).wait()
            col = jnp.full((16,), vids[k] % 128, jnp.int32)
            iv = jnp.full((16,), j * 16 + k, jnp.int32)
            top = plsc.load_gather(tiles.at[k], [lanes, col])
            bot = plsc.load_gather(tiles.at[k], [lanes + 16, col])
            plsc.store_scatter(colbuf, [lanes, iv], top)
            plsc.store_scatter(colbuf, [lanes + 16, iv], bot)
        return carry

    lax.fori_loop(0, _BPW // 16, group, 0)
    pltpu.sync_copy(colbuf, vecT_out.at[:, pl.ds(wb, _BPW)])


_vec_kernel = functools.partial(
    pl.kernel,
    mesh=plsc.VectorSubcoreMesh(core_axis_name="c", subcore_axis_name="s"),
    compiler_params=pltpu.CompilerParams(
        use_tc_tiling_on_sc=True, needs_layout_passes=False),
    out_type=jax.ShapeDtypeStruct((_D, _B), jnp.float32),
    scratch_types=[
        pltpu.VMEM((_B,), jnp.int32),
        pltpu.VMEM((16, _D, 128), jnp.float32),
        pltpu.VMEM((_D, _BPW), jnp.float32),
        pltpu.SemaphoreType.DMA,
    ],
)(_vec_body)


@jax.jit
def kernel(embedding, word_embedding0, lookup_ids, word_lookup_ids, random_ids):
    lid = lookup_ids.reshape(_N).astype(jnp.int32)
    rid = random_ids.reshape(_N).astype(jnp.int32)
    wid = word_lookup_ids.astype(jnp.int32)
    mat, real, eos, rnd = _sc_kernel(embedding, lid, rid)
    vecT = _vec_kernel(word_embedding0.T, wid)
    return (
        mat.reshape(_B, _L, _D),
        real.reshape(_B, _L),
        eos.reshape(_B, _L),
        vecT.T,
        rnd.reshape(_B, _L, _D),
    )
